# baseline (speedup 1.0000x reference)
import functools

import jax
import jax.numpy as jnp
from jax.experimental import pallas as pl
from jax.experimental.pallas import tpu as pltpu

DC = 8


def _body(NC, S, Dm, q_ref, k_ref, v_ref, wq_ref, wk_ref, wv_ref, wo_ref,
          bv_ref, bo_ref, bq_s, bk_s, ws_s, out_ref, attn_ref,
          qts, kts, vps, acc):
    f32 = jnp.float32
    c = pl.program_id(1)

    @pl.when(c == 0)
    def _proj():
        qts[...] = jax.lax.dot_general(
            wq_ref[...], q_ref[0], (((1,), (1,)), ((), ())),
            preferred_element_type=f32)
        kts[...] = jax.lax.dot_general(
            wk_ref[...], k_ref[0], (((1,), (1,)), ((), ())),
            preferred_element_type=f32)
        vps[...] = jax.lax.dot_general(
            v_ref[0], wv_ref[...], (((1,), (1,)), ((), ())),
            preferred_element_type=f32) + bv_ref[...]
        acc[...] = jnp.zeros((S, S), f32)

    base = pl.multiple_of(c * DC, DC)
    qtc = qts[pl.ds(base, DC), :]
    ktc = kts[pl.ds(base, DC), :]
    upd = None
    for r in range(DC):
        d = c * DC + r
        w_r = ws_s[0, d]
        b_r = bq_s[0, d] + bk_s[0, d]
        krow = ktc[r:r + 1, :] + b_r
        qcol = jnp.transpose(qtc[r:r + 1, :], (1, 0))
        term = w_r * jnp.tanh(qcol + krow)
        upd = term if upd is None else upd + term
    acc[...] = acc[...] + upd

    @pl.when(c == NC - 1)
    def _epi():
        sc = acc[...]
        m = jnp.max(sc, axis=1, keepdims=True)
        e = jnp.exp(sc - m)
        s = jnp.sum(e, axis=1, keepdims=True)
        p = e / s
        attn_ref[0, 0] = p
        av = jax.lax.dot_general(p, vps[...], (((1,), (0,)), ((), ())),
                                 preferred_element_type=f32)
        out_ref[0] = jax.lax.dot_general(
            av, wo_ref[...], (((1,), (1,)), ((), ())),
            preferred_element_type=f32) + bo_ref[...]


def _fwd(query, key, value, Wq, bq, Wk, bk, Wv, bv, Ws, bs, Wo, bo,
         interpret=False):
    f32 = jnp.float32
    B, S, Dm = query.shape
    NC = Dm // DC
    body = functools.partial(_body, NC, S, Dm)

    in_specs = [
        pl.BlockSpec((1, S, Dm), lambda b, c: (b, 0, 0)),
        pl.BlockSpec((1, S, Dm), lambda b, c: (b, 0, 0)),
        pl.BlockSpec((1, S, Dm), lambda b, c: (b, 0, 0)),
        pl.BlockSpec((Dm, Dm), lambda b, c: (0, 0)),
        pl.BlockSpec((Dm, Dm), lambda b, c: (0, 0)),
        pl.BlockSpec((Dm, Dm), lambda b, c: (0, 0)),
        pl.BlockSpec((Dm, Dm), lambda b, c: (0, 0)),
        pl.BlockSpec((1, Dm), lambda b, c: (0, 0)),
        pl.BlockSpec((1, Dm), lambda b, c: (0, 0)),
        pl.BlockSpec(memory_space=pltpu.SMEM),
        pl.BlockSpec(memory_space=pltpu.SMEM),
        pl.BlockSpec(memory_space=pltpu.SMEM),
    ]
    out_specs = [
        pl.BlockSpec((1, S, Dm), lambda b, c: (b, 0, 0)),
        pl.BlockSpec((1, 1, S, S), lambda b, c: (b, 0, 0, 0)),
    ]
    out_shape = [
        jax.ShapeDtypeStruct((B, S, Dm), f32),
        jax.ShapeDtypeStruct((B, 1, S, S), f32),
    ]
    scratch = [
        pltpu.VMEM((Dm, S), f32),
        pltpu.VMEM((Dm, S), f32),
        pltpu.VMEM((S, Dm), f32),
        pltpu.VMEM((S, S), f32),
    ]
    out, attn = pl.pallas_call(
        body,
        grid=(B, NC),
        in_specs=in_specs,
        out_specs=out_specs,
        out_shape=out_shape,
        scratch_shapes=scratch,
        compiler_params=pltpu.CompilerParams(
            dimension_semantics=("parallel", "arbitrary"),
            vmem_limit_bytes=48 * 1024 * 1024,
        ),
        interpret=interpret,
    )(query, key, value, Wq, Wk, Wv, Wo,
      bv.reshape(1, Dm), bo.reshape(1, Dm),
      bq.reshape(1, Dm), bk.reshape(1, Dm), Ws.reshape(1, Dm))
    return out, attn


def kernel(query, key, value, Wq, bq, Wk, bk, Wv, bv, Ws, bs, Wo, bo):
    return _fwd(query, key, value, Wq, bq, Wk, bk, Wv, bv, Ws, bs, Wo, bo)


# speedup vs baseline: 1.1997x; 1.1997x over previous
import functools

import jax
import jax.numpy as jnp
from jax.experimental import pallas as pl
from jax.experimental.pallas import tpu as pltpu

DC = 16


def _body(NC, S, Dm, q_ref, k_ref, v_ref, wq_ref, wk_ref, wv_ref, wo_ref,
          bqk_ref, bv_ref, bo_ref, ws_s, out_ref, attn_ref,
          qts, kts, vps, acc):
    f32 = jnp.float32
    bf16 = jnp.bfloat16
    c = pl.program_id(1)

    @pl.when(c == 0)
    def _proj():
        qts[...] = jax.lax.dot_general(
            wq_ref[...], q_ref[0], (((1,), (1,)), ((), ())),
            preferred_element_type=f32).astype(bf16)
        kts[...] = (jax.lax.dot_general(
            wk_ref[...], k_ref[0], (((1,), (1,)), ((), ())),
            preferred_element_type=f32) + bqk_ref[...]).astype(bf16)
        vps[...] = jax.lax.dot_general(
            v_ref[0], wv_ref[...], (((1,), (1,)), ((), ())),
            preferred_element_type=f32) + bv_ref[...]
        acc[...] = jnp.zeros((S, S), f32)

    base = pl.multiple_of(c * DC, DC)
    qtc = qts[pl.ds(base, DC), :]
    ktc = kts[pl.ds(base, DC), :]
    ws = [ws_s[0, c * DC + r].astype(bf16) for r in range(DC)]
    krows = [ktc[r:r + 1, :] for r in range(DC)]
    for i in range(DC // 2):
        r0, r1 = 2 * i, 2 * i + 1
        q0 = jnp.transpose(qtc[r0:r0 + 1, :], (1, 0))
        q1 = jnp.transpose(qtc[r1:r1 + 1, :], (1, 0))
        t0 = jnp.tanh(q0 + krows[r0]) * ws[r0]
        t1 = jnp.tanh(q1 + krows[r1]) * ws[r1]
        acc[...] = acc[...] + (t0 + t1).astype(f32)

    @pl.when(c == NC - 1)
    def _epi():
        sc = acc[...]
        m = jnp.max(sc, axis=1, keepdims=True)
        e = jnp.exp(sc - m)
        s = jnp.sum(e, axis=1, keepdims=True)
        p = e / s
        attn_ref[0, 0] = p
        av = jax.lax.dot_general(p, vps[...], (((1,), (0,)), ((), ())),
                                 preferred_element_type=f32)
        out_ref[0] = jax.lax.dot_general(
            av, wo_ref[...], (((1,), (1,)), ((), ())),
            preferred_element_type=f32) + bo_ref[...]


def _fwd(query, key, value, Wq, bq, Wk, bk, Wv, bv, Ws, bs, Wo, bo,
         interpret=False):
    f32 = jnp.float32
    B, S, Dm = query.shape
    NC = Dm // DC
    body = functools.partial(_body, NC, S, Dm)

    in_specs = [
        pl.BlockSpec((1, S, Dm), lambda b, c: (b, 0, 0)),
        pl.BlockSpec((1, S, Dm), lambda b, c: (b, 0, 0)),
        pl.BlockSpec((1, S, Dm), lambda b, c: (b, 0, 0)),
        pl.BlockSpec((Dm, Dm), lambda b, c: (0, 0)),
        pl.BlockSpec((Dm, Dm), lambda b, c: (0, 0)),
        pl.BlockSpec((Dm, Dm), lambda b, c: (0, 0)),
        pl.BlockSpec((Dm, Dm), lambda b, c: (0, 0)),
        pl.BlockSpec((Dm, 1), lambda b, c: (0, 0)),
        pl.BlockSpec((1, Dm), lambda b, c: (0, 0)),
        pl.BlockSpec((1, Dm), lambda b, c: (0, 0)),
        pl.BlockSpec(memory_space=pltpu.SMEM),
    ]
    out_specs = [
        pl.BlockSpec((1, S, Dm), lambda b, c: (b, 0, 0)),
        pl.BlockSpec((1, 1, S, S), lambda b, c: (b, 0, 0, 0)),
    ]
    out_shape = [
        jax.ShapeDtypeStruct((B, S, Dm), f32),
        jax.ShapeDtypeStruct((B, 1, S, S), f32),
    ]
    scratch = [
        pltpu.VMEM((Dm, S), jnp.bfloat16),
        pltpu.VMEM((Dm, S), jnp.bfloat16),
        pltpu.VMEM((S, Dm), f32),
        pltpu.VMEM((S, S), f32),
    ]
    out, attn = pl.pallas_call(
        body,
        grid=(B, NC),
        in_specs=in_specs,
        out_specs=out_specs,
        out_shape=out_shape,
        scratch_shapes=scratch,
        compiler_params=pltpu.CompilerParams(
            dimension_semantics=("parallel", "arbitrary"),
            vmem_limit_bytes=48 * 1024 * 1024,
        ),
        interpret=interpret,
    )(query, key, value, Wq, Wk, Wv, Wo,
      (bq + bk).reshape(Dm, 1), bv.reshape(1, Dm), bo.reshape(1, Dm),
      Ws.reshape(1, Dm))
    return out, attn


def kernel(query, key, value, Wq, bq, Wk, bk, Wv, bv, Ws, bs, Wo, bo):
    return _fwd(query, key, value, Wq, bq, Wk, bk, Wv, bv, Ws, bs, Wo, bo)


# speedup vs baseline: 1.6841x; 1.4038x over previous
import functools

import jax
import jax.numpy as jnp
from jax.experimental import pallas as pl
from jax.experimental.pallas import tpu as pltpu

IB = 128


def _body(NI, S, Dm, q_ref, k_ref, v_ref, wq_ref, wk_ref, wv_ref, wo_ref,
          bqk_ref, bv_ref, bo_ref, w2_ref, out_ref, attn_ref,
          qts3, kts, vps, sc):
    f32 = jnp.float32
    bf16 = jnp.bfloat16
    c = pl.program_id(1)

    @pl.when(c == 0)
    def _proj():
        for n in range(NI):
            qts3[n] = jax.lax.dot_general(
                wq_ref[...], q_ref[0, n * IB:(n + 1) * IB, :],
                (((1,), (1,)), ((), ())),
                preferred_element_type=f32).astype(bf16)
        kts[...] = (jax.lax.dot_general(
            wk_ref[...], k_ref[0], (((1,), (1,)), ((), ())),
            preferred_element_type=f32) + bqk_ref[...]).astype(bf16)
        vps[...] = jax.lax.dot_general(
            v_ref[0], wv_ref[...], (((1,), (1,)), ((), ())),
            preferred_element_type=f32) + bv_ref[...]

    w2b = w2_ref[...].astype(bf16)
    qtb = qts3[c]
    kf = kts[...]
    for g in range(IB // 8):
        slabs = [jnp.tanh(qtb[:, g * 8 + m:g * 8 + m + 1] + kf)
                 for m in range(8)]
        t8 = jnp.concatenate(slabs, axis=0)
        sc[g * 8:(g + 1) * 8, :] = jax.lax.dot_general(
            w2b, t8, (((1,), (0,)), ((), ())), preferred_element_type=f32)

    s = sc[...]
    m = jnp.max(s, axis=1, keepdims=True)
    e = jnp.exp(s - m)
    den = jnp.sum(e, axis=1, keepdims=True)
    p = e / den
    attn_ref[0, 0] = p
    av = jax.lax.dot_general(p, vps[...], (((1,), (0,)), ((), ())),
                             preferred_element_type=f32)
    out_ref[0] = jax.lax.dot_general(
        av, wo_ref[...], (((1,), (1,)), ((), ())),
        preferred_element_type=f32) + bo_ref[...]


def _fwd(query, key, value, Wq, bq, Wk, bk, Wv, bv, Ws, bs, Wo, bo,
         interpret=False):
    f32 = jnp.float32
    B, S, Dm = query.shape
    NI = S // IB
    body = functools.partial(_body, NI, S, Dm)

    in_specs = [
        pl.BlockSpec((1, S, Dm), lambda b, c: (b, 0, 0)),
        pl.BlockSpec((1, S, Dm), lambda b, c: (b, 0, 0)),
        pl.BlockSpec((1, S, Dm), lambda b, c: (b, 0, 0)),
        pl.BlockSpec((Dm, Dm), lambda b, c: (0, 0)),
        pl.BlockSpec((Dm, Dm), lambda b, c: (0, 0)),
        pl.BlockSpec((Dm, Dm), lambda b, c: (0, 0)),
        pl.BlockSpec((Dm, Dm), lambda b, c: (0, 0)),
        pl.BlockSpec((Dm, 1), lambda b, c: (0, 0)),
        pl.BlockSpec((1, Dm), lambda b, c: (0, 0)),
        pl.BlockSpec((1, Dm), lambda b, c: (0, 0)),
        pl.BlockSpec((8, 8 * Dm), lambda b, c: (0, 0)),
    ]
    out_specs = [
        pl.BlockSpec((1, IB, Dm), lambda b, c: (b, c, 0)),
        pl.BlockSpec((1, 1, IB, S), lambda b, c: (b, 0, c, 0)),
    ]
    out_shape = [
        jax.ShapeDtypeStruct((B, S, Dm), f32),
        jax.ShapeDtypeStruct((B, 1, S, S), f32),
    ]
    scratch = [
        pltpu.VMEM((NI, Dm, IB), jnp.bfloat16),
        pltpu.VMEM((Dm, S), jnp.bfloat16),
        pltpu.VMEM((S, Dm), f32),
        pltpu.VMEM((IB, S), f32),
    ]
    w2 = jnp.kron(jnp.eye(8, dtype=f32), Ws.reshape(1, Dm))
    out, attn = pl.pallas_call(
        body,
        grid=(B, NI),
        in_specs=in_specs,
        out_specs=out_specs,
        out_shape=out_shape,
        scratch_shapes=scratch,
        compiler_params=pltpu.CompilerParams(
            dimension_semantics=("parallel", "arbitrary"),
            vmem_limit_bytes=48 * 1024 * 1024,
        ),
        interpret=interpret,
    )(query, key, value, Wq, Wk, Wv, Wo,
      (bq + bk).reshape(Dm, 1), bv.reshape(1, Dm), bo.reshape(1, Dm), w2)
    return out, attn


def kernel(query, key, value, Wq, bq, Wk, bk, Wv, bv, Ws, bs, Wo, bo):
    return _fwd(query, key, value, Wq, bq, Wk, bk, Wv, bv, Ws, bs, Wo, bo)
